# Initial kernel scaffold; baseline (speedup 1.0000x reference)
import jax
import jax.numpy as jnp
from jax.experimental import pallas as pl


def kernel(weight, alpha, flip_idx):
    raise NotImplementedError("write your pallas kernel here")


# speedup vs baseline: 2.0873x; 2.0873x over previous
import functools
import jax
import jax.numpy as jnp
from jax import lax
from jax.experimental import pallas as pl
from jax.experimental.pallas import tpu as pltpu
from jax.experimental.pallas import tpu_sc as plsc

_N_BITS = 8
_QN = float(-(2 ** (_N_BITS - 1)) + 1)
_QP = float(2 ** (_N_BITS - 1) - 1)
_XOR = 1 << (_N_BITS - 1)

_NC, _NS, _L = 2, 16, 16
_NW = _NC * _NS
_CHUNK = 16384

_MAGIC = 12582912.0


@functools.lru_cache(maxsize=None)
def _build(n_flat, flip_pad):
  per_tile = n_flat // _NW
  nchunk = per_tile // _CHUNK
  n_fvec = flip_pad // _L
  mesh = plsc.VectorSubcoreMesh(
      core_axis_name="core", subcore_axis_name="subcore",
      num_cores=_NC, num_subcores=_NS)

  @functools.partial(
      pl.kernel,
      out_type=jax.ShapeDtypeStruct((n_flat,), jnp.float32),
      mesh=mesh,
      compiler_params=pltpu.CompilerParams(needs_layout_passes=False),
      scratch_types=[
          pltpu.VMEM((_CHUNK,), jnp.float32),
          pltpu.VMEM((_CHUNK,), jnp.float32),
          pltpu.VMEM((_CHUNK,), jnp.float32),
          pltpu.VMEM((_CHUNK,), jnp.float32),
          pltpu.VMEM((flip_pad,), jnp.int32),
          pltpu.VMEM((_L,), jnp.float32),
          pltpu.SemaphoreType.DMA,
          pltpu.SemaphoreType.DMA,
          pltpu.SemaphoreType.DMA,
          pltpu.SemaphoreType.DMA,
      ],
  )
  def launch(w_hbm, alpha_hbm, fidx_hbm, out_hbm,
             in0, in1, o0, o1, idx_v, alpha_ref,
             isem0, isem1, osem0, osem1):
    wid = lax.axis_index("subcore") * _NC + lax.axis_index("core")
    base_t = wid * per_tile

    pltpu.sync_copy(fidx_hbm, idx_v)
    pltpu.sync_copy(alpha_hbm, alpha_ref)
    alpha_v = jnp.maximum(alpha_ref[...], 1e-4)
    inv_v = 1.0 / alpha_v

    ins = (in0, in1)
    outs = (o0, o1)
    isems = (isem0, isem1)
    osems = (osem0, osem1)
    in_d = [None] * nchunk
    out_d = [None] * nchunk

    in_d[0] = pltpu.async_copy(
        w_hbm.at[pl.ds(base_t, _CHUNK)], ins[0], isems[0])

    for c in range(nchunk):
      cur = c & 1
      base = base_t + c * _CHUNK
      if c + 1 < nchunk:
        in_d[c + 1] = pltpu.async_copy(
            w_hbm.at[pl.ds(base_t + (c + 1) * _CHUNK, _CHUNK)],
            ins[(c + 1) & 1], isems[(c + 1) & 1])
      in_d[c].wait()
      if c >= 2:
        out_d[c - 2].wait()

      in_ref = ins[cur]
      out_ref = outs[cur]

      @plsc.parallel_loop(0, _CHUNK, step=_L, unroll=8)
      def _dense(i):
        x = in_ref[pl.ds(i, _L)]
        q = jnp.minimum(jnp.maximum(x * inv_v, _QN), _QP)
        r = (q + _MAGIC) - _MAGIC
        out_ref[pl.ds(i, _L)] = r * alpha_v

      @plsc.parallel_loop(0, flip_pad, step=_L)
      def _flips(j):
        iv = idx_v[pl.ds(j, _L)]
        m = (iv >= base) & (iv < base + _CHUNK)
        loc = jnp.minimum(jnp.maximum(iv - base, 0), _CHUNK - 1)
        wv = plsc.load_gather(in_ref, [loc], mask=m)
        q = jnp.minimum(jnp.maximum(wv * inv_v, _QN), _QP)
        t = q.astype(jnp.int32) ^ _XOR
        plsc.store_scatter(out_ref, [loc], t.astype(jnp.float32) * alpha_v,
                           mask=m)

      out_d[c] = pltpu.async_copy(
          out_ref, out_hbm.at[pl.ds(base, _CHUNK)], osems[cur])

    out_d[nchunk - 2].wait()
    out_d[nchunk - 1].wait()

  return launch


def kernel(weight, alpha, flip_idx):
  n = weight.size
  fn = flip_idx.shape[0]
  fp = -(-fn // _L) * _L
  fidx = flip_idx.astype(jnp.int32)
  if fp > fn:
    fidx = jnp.concatenate(
        [fidx, jnp.full((fp - fn,), -1, dtype=jnp.int32)])
  alpha16 = jnp.broadcast_to(
      alpha.astype(jnp.float32).reshape(()), (_L,))
  out = _build(n, fp)(weight.reshape(-1), alpha16, fidx)
  return out.reshape(weight.shape)


# speedup vs baseline: 2.1388x; 1.0247x over previous
import functools
import jax
import jax.numpy as jnp
from jax import lax
from jax.experimental import pallas as pl
from jax.experimental.pallas import tpu as pltpu
from jax.experimental.pallas import tpu_sc as plsc

_N_BITS = 8
_QN = float(-(2 ** (_N_BITS - 1)) + 1)
_QP = float(2 ** (_N_BITS - 1) - 1)
_XOR = 1 << (_N_BITS - 1)

_NC, _NS, _L = 2, 16, 16
_NW = _NC * _NS
_CHUNK = 16384

_MAGIC = 12582912.0


@functools.lru_cache(maxsize=None)
def _build(n_flat, flip_pad):
  per_tile = n_flat // _NW
  nchunk = per_tile // _CHUNK
  n_fvec = flip_pad // _L
  mesh = plsc.VectorSubcoreMesh(
      core_axis_name="core", subcore_axis_name="subcore",
      num_cores=_NC, num_subcores=_NS)

  @functools.partial(
      pl.kernel,
      out_type=jax.ShapeDtypeStruct((n_flat,), jnp.float32),
      mesh=mesh,
      compiler_params=pltpu.CompilerParams(needs_layout_passes=False),
      scratch_types=[
          pltpu.VMEM((_CHUNK,), jnp.float32),
          pltpu.VMEM((_CHUNK,), jnp.float32),
          pltpu.VMEM((_CHUNK,), jnp.float32),
          pltpu.VMEM((_CHUNK,), jnp.float32),
          pltpu.VMEM((flip_pad,), jnp.int32),
          pltpu.VMEM((flip_pad + _L,), jnp.int32),
          pltpu.VMEM((_L,), jnp.float32),
          pltpu.SemaphoreType.DMA,
          pltpu.SemaphoreType.DMA,
          pltpu.SemaphoreType.DMA,
          pltpu.SemaphoreType.DMA,
      ],
  )
  def launch(w_hbm, alpha_hbm, fidx_hbm, out_hbm,
             in0, in1, o0, o1, idx_v, tidx_v, alpha_ref,
             isem0, isem1, osem0, osem1):
    wid = lax.axis_index("subcore") * _NC + lax.axis_index("core")
    base_t = wid * per_tile

    pltpu.sync_copy(fidx_hbm, idx_v)
    pltpu.sync_copy(alpha_hbm, alpha_ref)
    alpha_v = jnp.maximum(alpha_ref[...], 1e-4)
    inv_v = 1.0 / alpha_v

    def _compact(j, cnt):
      iv = idx_v[pl.ds(j * _L, _L)]
      m = (iv >= base_t) & (iv < base_t + per_tile)
      plsc.store_compressed(tidx_v.at[pl.ds(cnt, _L)], iv - base_t, mask=m)
      return cnt + jnp.sum(m.astype(jnp.int32))

    cnt = lax.fori_loop(0, n_fvec, _compact, jnp.int32(0))
    n_tvec = (cnt + _L - 1) // _L
    lane = lax.iota(jnp.int32, _L)

    ins = (in0, in1)
    outs = (o0, o1)
    isems = (isem0, isem1)
    osems = (osem0, osem1)
    in_d = [None] * nchunk
    out_d = [None] * nchunk

    in_d[0] = pltpu.async_copy(
        w_hbm.at[pl.ds(base_t, _CHUNK)], ins[0], isems[0])

    for c in range(nchunk):
      cur = c & 1
      base = base_t + c * _CHUNK
      if c + 1 < nchunk:
        in_d[c + 1] = pltpu.async_copy(
            w_hbm.at[pl.ds(base_t + (c + 1) * _CHUNK, _CHUNK)],
            ins[(c + 1) & 1], isems[(c + 1) & 1])
      in_d[c].wait()
      if c >= 2:
        out_d[c - 2].wait()

      in_ref = ins[cur]
      out_ref = outs[cur]

      @plsc.parallel_loop(0, _CHUNK, step=_L, unroll=8)
      def _dense(i):
        x = in_ref[pl.ds(i, _L)]
        q = jnp.minimum(jnp.maximum(x * inv_v, _QN), _QP)
        r = (q + _MAGIC) - _MAGIC
        out_ref[pl.ds(i, _L)] = r * alpha_v

      cbase = c * _CHUNK

      def _flips(j, _):
        lv = tidx_v[pl.ds(j * _L, _L)]
        m = ((j * _L + lane < cnt)
             & (lv >= cbase) & (lv < cbase + _CHUNK))
        loc = jnp.minimum(jnp.maximum(lv - cbase, 0), _CHUNK - 1)
        wv = plsc.load_gather(in_ref, [loc], mask=m)
        q = jnp.minimum(jnp.maximum(wv * inv_v, _QN), _QP)
        t = q.astype(jnp.int32) ^ _XOR
        plsc.store_scatter(out_ref, [loc], t.astype(jnp.float32) * alpha_v,
                           mask=m)
        return 0

      lax.fori_loop(0, n_tvec, _flips, 0)

      out_d[c] = pltpu.async_copy(
          out_ref, out_hbm.at[pl.ds(base, _CHUNK)], osems[cur])

    out_d[nchunk - 2].wait()
    out_d[nchunk - 1].wait()

  return launch


def kernel(weight, alpha, flip_idx):
  n = weight.size
  fn = flip_idx.shape[0]
  fp = -(-fn // _L) * _L
  fidx = flip_idx.astype(jnp.int32)
  if fp > fn:
    fidx = jnp.concatenate(
        [fidx, jnp.full((fp - fn,), -1, dtype=jnp.int32)])
  alpha16 = jnp.broadcast_to(
      alpha.astype(jnp.float32).reshape(()), (_L,))
  out = _build(n, fp)(weight.reshape(-1), alpha16, fidx)
  return out.reshape(weight.shape)


# speedup vs baseline: 5.1095x; 2.3889x over previous
import functools
import jax
import jax.numpy as jnp
from jax import lax
from jax.experimental import pallas as pl
from jax.experimental.pallas import tpu as pltpu
from jax.experimental.pallas import tpu_sc as plsc

_N_BITS = 8
_QN = float(-(2 ** (_N_BITS - 1)) + 1)
_QP = float(2 ** (_N_BITS - 1) - 1)
_XOR = 1 << (_N_BITS - 1)

_NC, _NS, _L = 2, 16, 16
_NW = _NC * _NS
_NR = 4

_MAGIC = 12582912.0


@functools.lru_cache(maxsize=None)
def _build(nrow, ncol, flip_pad):
  rows_per_tile = nrow // _NW
  nchunk = rows_per_tile // _NR
  chunk_elems = _NR * ncol
  n_fvec = flip_pad // _L
  col_shift = ncol.bit_length() - 1
  assert 1 << col_shift == ncol
  mesh = plsc.VectorSubcoreMesh(
      core_axis_name="core", subcore_axis_name="subcore",
      num_cores=_NC, num_subcores=_NS)

  @functools.partial(
      pl.kernel,
      out_type=jax.ShapeDtypeStruct((nrow, ncol), jnp.float32),
      mesh=mesh,
      compiler_params=pltpu.CompilerParams(needs_layout_passes=False),
      scratch_types=[
          pltpu.VMEM((_NR, ncol), jnp.float32),
          pltpu.VMEM((_NR, ncol), jnp.float32),
          pltpu.VMEM((_NR, ncol), jnp.float32),
          pltpu.VMEM((_NR, ncol), jnp.float32),
          pltpu.VMEM((flip_pad,), jnp.int32),
          pltpu.VMEM((flip_pad + _L,), jnp.int32),
          pltpu.VMEM((_L,), jnp.float32),
          pltpu.SemaphoreType.DMA,
          pltpu.SemaphoreType.DMA,
          pltpu.SemaphoreType.DMA,
          pltpu.SemaphoreType.DMA,
      ],
  )
  def launch(w_hbm, alpha_hbm, fidx_hbm, out_hbm,
             in0, in1, o0, o1, idx_v, tidx_v, alpha_ref,
             isem0, isem1, osem0, osem1):
    wid = lax.axis_index("subcore") * _NC + lax.axis_index("core")
    row_t = wid * rows_per_tile
    base_t = row_t * ncol

    pltpu.sync_copy(fidx_hbm, idx_v)
    pltpu.sync_copy(alpha_hbm, alpha_ref)
    alpha_v = jnp.maximum(alpha_ref[...], 1e-4)
    inv_v = 1.0 / alpha_v

    def _compact(j, cnt):
      iv = idx_v[pl.ds(j * _L, _L)]
      m = (iv >= base_t) & (iv < base_t + rows_per_tile * ncol)
      plsc.store_compressed(tidx_v.at[pl.ds(cnt, _L)], iv - base_t, mask=m)
      return cnt + jnp.sum(m.astype(jnp.int32))

    cnt = lax.fori_loop(0, n_fvec, _compact, jnp.int32(0))
    n_tvec = (cnt + _L - 1) // _L
    lane = lax.iota(jnp.int32, _L)

    ins = (in0, in1)
    outs = (o0, o1)
    isems = (isem0, isem1)
    osems = (osem0, osem1)
    in_d = [None] * nchunk
    out_d = [None] * nchunk

    in_d[0] = pltpu.async_copy(
        w_hbm.at[pl.ds(row_t, _NR), :], ins[0], isems[0])

    for c in range(nchunk):
      cur = c & 1
      row = row_t + c * _NR
      if c + 1 < nchunk:
        in_d[c + 1] = pltpu.async_copy(
            w_hbm.at[pl.ds(row + _NR, _NR), :],
            ins[(c + 1) & 1], isems[(c + 1) & 1])
      in_d[c].wait()
      if c >= 2:
        out_d[c - 2].wait()

      in_ref = ins[cur]
      out_ref = outs[cur]

      @plsc.parallel_loop(0, _NR, step=1)
      def _rows(rr):
        @plsc.parallel_loop(0, ncol, step=_L, unroll=8)
        def _dense(i):
          x = in_ref[rr, pl.ds(i, _L)]
          q = jnp.minimum(jnp.maximum(x * inv_v, _QN), _QP)
          r = (q + _MAGIC) - _MAGIC
          out_ref[rr, pl.ds(i, _L)] = r * alpha_v

      cbase = c * chunk_elems

      def _flips(j, _):
        lv = tidx_v[pl.ds(j * _L, _L)]
        m = ((j * _L + lane < cnt)
             & (lv >= cbase) & (lv < cbase + chunk_elems))
        loc = jnp.minimum(jnp.maximum(lv - cbase, 0), chunk_elems - 1)
        loc_r = lax.shift_right_logical(loc, col_shift)
        loc_c = loc & (ncol - 1)
        wv = plsc.load_gather(in_ref, [loc_r, loc_c], mask=m)
        q = jnp.minimum(jnp.maximum(wv * inv_v, _QN), _QP)
        t = q.astype(jnp.int32) ^ _XOR
        plsc.store_scatter(out_ref, [loc_r, loc_c],
                           t.astype(jnp.float32) * alpha_v, mask=m)
        return 0

      lax.fori_loop(0, n_tvec, _flips, 0)

      out_d[c] = pltpu.async_copy(
          out_ref, out_hbm.at[pl.ds(row, _NR), :], osems[cur])

    out_d[nchunk - 2].wait()
    out_d[nchunk - 1].wait()

  return launch


def kernel(weight, alpha, flip_idx):
  nrow, ncol = weight.shape
  fn = flip_idx.shape[0]
  fp = -(-fn // _L) * _L
  fidx = flip_idx.astype(jnp.int32)
  if fp > fn:
    fidx = jnp.concatenate(
        [fidx, jnp.full((fp - fn,), -1, dtype=jnp.int32)])
  alpha16 = jnp.broadcast_to(
      alpha.astype(jnp.float32).reshape(()), (_L,))
  return _build(nrow, ncol, fp)(weight, alpha16, fidx)


# speedup vs baseline: 5.1304x; 1.0041x over previous
import functools
import jax
import jax.numpy as jnp
from jax import lax
from jax.experimental import pallas as pl
from jax.experimental.pallas import tpu as pltpu
from jax.experimental.pallas import tpu_sc as plsc

_N_BITS = 8
_QN = float(-(2 ** (_N_BITS - 1)) + 1)
_QP = float(2 ** (_N_BITS - 1) - 1)
_XOR = 1 << (_N_BITS - 1)

_NC, _NS, _L = 2, 16, 16
_NW = _NC * _NS
_NR = 4

_MAGIC = 12582912.0


@functools.lru_cache(maxsize=None)
def _build(nrow, ncol, n_flip):
  rows_per_tile = nrow // _NW
  nchunk = rows_per_tile // _NR
  chunk_elems = _NR * ncol
  n_fvec = -(-n_flip // _L)
  col_shift = ncol.bit_length() - 1
  assert 1 << col_shift == ncol
  mesh = plsc.VectorSubcoreMesh(
      core_axis_name="core", subcore_axis_name="subcore",
      num_cores=_NC, num_subcores=_NS)

  @functools.partial(
      pl.kernel,
      out_type=jax.ShapeDtypeStruct((nrow, ncol), jnp.float32),
      mesh=mesh,
      compiler_params=pltpu.CompilerParams(needs_layout_passes=False),
      scratch_types=[
          pltpu.VMEM((_NR, ncol), jnp.float32),
          pltpu.VMEM((_NR, ncol), jnp.float32),
          pltpu.VMEM((_NR, ncol), jnp.float32),
          pltpu.VMEM((_NR, ncol), jnp.float32),
          pltpu.VMEM((n_flip,), jnp.int32),
          pltpu.VMEM((n_flip + _L,), jnp.int32),
          pltpu.VMEM((_L,), jnp.float32),
          pltpu.SemaphoreType.DMA,
          pltpu.SemaphoreType.DMA,
          pltpu.SemaphoreType.DMA,
          pltpu.SemaphoreType.DMA,
      ],
  )
  def launch(w_hbm, alpha_hbm, fidx_hbm, out_hbm,
             in0, in1, o0, o1, idx_v, tidx_v, alpha_ref,
             isem0, isem1, osem0, osem1):
    wid = lax.axis_index("subcore") * _NC + lax.axis_index("core")
    row_t = wid * rows_per_tile
    base_t = row_t * ncol

    ins = (in0, in1)
    outs = (o0, o1)
    isems = (isem0, isem1)
    osems = (osem0, osem1)
    in_d = [None] * nchunk
    out_d = [None] * nchunk

    in_d[0] = pltpu.async_copy(
        w_hbm.at[pl.ds(row_t, _NR), :], ins[0], isems[0])

    pltpu.sync_copy(fidx_hbm, idx_v)
    pltpu.sync_copy(alpha_hbm, alpha_ref.at[pl.ds(0, 1)])
    lane = lax.iota(jnp.int32, _L)
    alpha_v = jnp.maximum(plsc.load_gather(alpha_ref, [lane * 0]), 1e-4)
    inv_v = 1.0 / alpha_v

    def _compact(j, cnt):
      start = jnp.maximum(jnp.minimum(j * _L, n_flip - _L), 0)
      gid = start + lane
      iv = idx_v[pl.ds(start, _L)]
      m = ((gid >= j * _L)
           & (iv >= base_t) & (iv < base_t + rows_per_tile * ncol))
      plsc.store_compressed(tidx_v.at[pl.ds(cnt, _L)], iv - base_t, mask=m)
      return cnt + jnp.sum(m.astype(jnp.int32))

    cnt = lax.fori_loop(0, n_fvec, _compact, jnp.int32(0))
    n_tvec = (cnt + _L - 1) // _L

    for c in range(nchunk):
      cur = c & 1
      row = row_t + c * _NR
      if c + 1 < nchunk:
        in_d[c + 1] = pltpu.async_copy(
            w_hbm.at[pl.ds(row + _NR, _NR), :],
            ins[(c + 1) & 1], isems[(c + 1) & 1])
      in_d[c].wait()
      if c >= 2:
        out_d[c - 2].wait()

      in_ref = ins[cur]
      out_ref = outs[cur]

      @plsc.parallel_loop(0, _NR, step=1)
      def _rows(rr):
        @plsc.parallel_loop(0, ncol, step=_L, unroll=8)
        def _dense(i):
          x = in_ref[rr, pl.ds(i, _L)]
          q = jnp.minimum(jnp.maximum(x * inv_v, _QN), _QP)
          r = (q + _MAGIC) - _MAGIC
          out_ref[rr, pl.ds(i, _L)] = r * alpha_v

      cbase = c * chunk_elems

      def _flips(j, _):
        lv = tidx_v[pl.ds(j * _L, _L)]
        m = ((j * _L + lane < cnt)
             & (lv >= cbase) & (lv < cbase + chunk_elems))
        loc = jnp.minimum(jnp.maximum(lv - cbase, 0), chunk_elems - 1)
        loc_r = lax.shift_right_logical(loc, col_shift)
        loc_c = loc & (ncol - 1)
        wv = plsc.load_gather(in_ref, [loc_r, loc_c], mask=m)
        q = jnp.minimum(jnp.maximum(wv * inv_v, _QN), _QP)
        t = q.astype(jnp.int32) ^ _XOR
        plsc.store_scatter(out_ref, [loc_r, loc_c],
                           t.astype(jnp.float32) * alpha_v, mask=m)
        return 0

      lax.fori_loop(0, n_tvec, _flips, 0)

      out_d[c] = pltpu.async_copy(
          out_ref, out_hbm.at[pl.ds(row, _NR), :], osems[cur])

    out_d[nchunk - 2].wait()
    out_d[nchunk - 1].wait()

  return launch


def kernel(weight, alpha, flip_idx):
  nrow, ncol = weight.shape
  return _build(nrow, ncol, flip_idx.shape[0])(
      weight, alpha.astype(jnp.float32), flip_idx.astype(jnp.int32))
